# baseline (speedup 1.0000x reference)
import functools

import jax
import jax.numpy as jnp
from jax import lax
from jax.experimental import pallas as pl
from jax.experimental.pallas import tpu as pltpu
from jax.experimental.pallas import tpu_sc as plsc

N = 10000
E = 320000
F = 128

NC = 2
NS = 16
NW = NC * NS

K = 128
NPAD = 10240
ROWS_PER_TILE = NPAD // NS
CW = 80
EW = CW * K
E_PAD = EW * NW
C_TOT = E_PAD // K

_mesh = plsc.VectorSubcoreMesh(core_axis_name="c", subcore_axis_name="s")


def _zero_vmem_1d(ref, n):
    z = jnp.zeros((16,), jnp.float32)

    def body(i, _):
        ref[pl.ds(i * 16, 16)] = z
        return 0

    lax.fori_loop(0, n // 16, body, 0)


def _zero_vmem_2d(ref, rows, cols):
    z = jnp.zeros((16,), jnp.float32)

    def body(i, _):
        r = i // (cols // 16)
        c = lax.rem(i, cols // 16) * 16
        ref[r, pl.ds(c, 16)] = z
        return 0

    lax.fori_loop(0, rows * (cols // 16), body, 0)


@functools.partial(
    pl.kernel,
    out_type=jax.ShapeDtypeStruct((NC, 2, NPAD), jnp.float32),
    mesh=_mesh,
    scratch_types=[
        pltpu.VMEM((CW, K), jnp.int32),
        pltpu.VMEM((CW, K), jnp.int32),
        pltpu.VMEM((K,), jnp.float32),
        pltpu.VMEM((NPAD // NS,), jnp.float32),
        pltpu.VMEM_SHARED((NPAD,), jnp.float32),
        pltpu.VMEM_SHARED((NPAD,), jnp.float32),
        pltpu.SemaphoreType.DMA,
    ],
)
def _sc_degrees(src_hbm, dst_hbm, deg_out_hbm, sidx_v, didx_v, ones_v,
                stage_v, dout_sh, din_sh, sem):
    cid = lax.axis_index("c")
    sid = lax.axis_index("s")
    wid = sid * NC + cid
    seg = NPAD // NS

    pltpu.async_copy(src_hbm.at[pl.ds(wid * CW, CW)], sidx_v, sem)
    pltpu.async_copy(dst_hbm.at[pl.ds(wid * CW, CW)], didx_v, sem)

    _zero_vmem_1d(stage_v, seg)
    pltpu.sync_copy(stage_v, dout_sh.at[pl.ds(sid * seg, seg)])
    pltpu.sync_copy(stage_v, din_sh.at[pl.ds(sid * seg, seg)])
    one = jnp.ones((16,), jnp.float32)
    for i in range(K // 16):
        ones_v[pl.ds(i * 16, 16)] = one
    pltpu.make_async_copy(src_hbm.at[pl.ds(0, CW)], sidx_v, sem).wait()
    pltpu.make_async_copy(src_hbm.at[pl.ds(0, CW)], didx_v, sem).wait()
    plsc.subcore_barrier()

    BATCH = 16

    def batch(b, _):
        def fire(j, _):
            pltpu.async_copy(ones_v, dout_sh.at[sidx_v.at[b * BATCH + j]],
                             sem, add=True)
            pltpu.async_copy(ones_v, din_sh.at[didx_v.at[b * BATCH + j]],
                             sem, add=True)
            return 0

        lax.fori_loop(0, BATCH, fire, 0)

        def drain(j, _):
            pltpu.make_async_copy(ones_v, dout_sh.at[sidx_v.at[0]], sem).wait()
            pltpu.make_async_copy(ones_v, din_sh.at[didx_v.at[0]], sem).wait()
            return 0

        lax.fori_loop(0, BATCH, drain, 0)
        return 0

    lax.fori_loop(0, CW // BATCH, batch, 0)
    plsc.subcore_barrier()

    pltpu.sync_copy(dout_sh.at[pl.ds(sid * seg, seg)], stage_v)
    pltpu.sync_copy(stage_v, deg_out_hbm.at[cid, 0, pl.ds(sid * seg, seg)])
    pltpu.sync_copy(din_sh.at[pl.ds(sid * seg, seg)], stage_v)
    pltpu.sync_copy(stage_v, deg_out_hbm.at[cid, 1, pl.ds(sid * seg, seg)])


def _mm_scale_body(x_ref, w_ref, deg_ref, out_ref):
    deg = deg_ref[0, :] + deg_ref[2, :]
    inv = lax.rsqrt(jnp.clip(deg, 1.0, None))
    out_ref[...] = jnp.dot(x_ref[...], w_ref[...],
                           preferred_element_type=jnp.float32) * inv[:, None]


_MM_BLK = 1280


def _mm_scale(x_pad, w1, deg):
    return pl.pallas_call(
        _mm_scale_body,
        grid=(NPAD // _MM_BLK,),
        in_specs=[
            pl.BlockSpec((_MM_BLK, F), lambda i: (i, 0)),
            pl.BlockSpec((F, F), lambda i: (0, 0)),
            pl.BlockSpec((2 * NC, _MM_BLK), lambda i: (0, i)),
        ],
        out_specs=pl.BlockSpec((_MM_BLK, F), lambda i: (i, 0)),
        out_shape=jax.ShapeDtypeStruct((NPAD, F), jnp.float32),
    )(x_pad, w1, deg)


IB = 8
NB0 = 16
NB1 = 4
NBMAX = max(NB0, NB1)
assert NS * (NB0 + NB1) * IB == C_TOT


@functools.partial(
    pl.kernel,
    out_type=jax.ShapeDtypeStruct((NC, NPAD, F), jnp.float32),
    mesh=_mesh,
    scratch_types=[
        pltpu.VMEM((IB, K), jnp.int32),
        pltpu.VMEM((IB, K), jnp.int32),
        pltpu.VMEM((IB, K), jnp.int32),
        pltpu.VMEM((IB, K), jnp.int32),
        pltpu.VMEM((K, F), jnp.float32),
        pltpu.VMEM((K, F), jnp.float32),
        pltpu.VMEM_SHARED((NPAD, F), jnp.float32),
        pltpu.SemaphoreType.DMA,
        pltpu.SemaphoreType.DMA,
        pltpu.SemaphoreType.DMA,
    ],
)
def _sc_edge_pass(sup_hbm, src_hbm, dst_hbm, agg_hbm, sidxA, didxA, sidxB,
                  didxB, rows0, rows1, agg_sh, sem0, sem1, semi):
    cid = lax.axis_index("c")
    sid = lax.axis_index("s")
    nb = jnp.where(cid == 0, NB0, NB1)
    cbase = cid * (NS * NB0 * IB) + sid * nb * IB

    def load_idx(b, sbuf, dbuf):
        pltpu.async_copy(src_hbm.at[pl.ds(cbase + b * IB, IB)], sbuf, semi)
        pltpu.async_copy(dst_hbm.at[pl.ds(cbase + b * IB, IB)], dbuf, semi)

    def wait_idx(sbuf, dbuf):
        pltpu.make_async_copy(src_hbm.at[pl.ds(0, IB)], sbuf, semi).wait()
        pltpu.make_async_copy(src_hbm.at[pl.ds(0, IB)], dbuf, semi).wait()

    load_idx(0, sidxA, didxA)
    load_idx(1, sidxB, didxB)

    _zero_vmem_2d(rows0, K, F)
    for r in range(ROWS_PER_TILE // K):
        pltpu.sync_copy(rows0, agg_sh.at[pl.ds(sid * ROWS_PER_TILE + r * K, K)])
    plsc.subcore_barrier()

    def gather(idx_row, buf, sem):
        pltpu.async_copy(sup_hbm.at[idx_row], buf, sem)

    def gwait(buf, sem):
        pltpu.make_async_copy(sup_hbm.at[pl.ds(0, K)], buf, sem).wait()

    def scat(idx_row, buf):
        pltpu.sync_copy(buf, agg_sh.at[idx_row], add=True)

    for b in range(NBMAX):
        sbuf, dbuf = (sidxA, didxA) if b % 2 == 0 else (sidxB, didxB)

        @pl.when(b < nb)
        def _(b=b, sbuf=sbuf, dbuf=dbuf):
            wait_idx(sbuf, dbuf)
            gather(sbuf.at[0], rows0, sem0)

            def body(i, _):
                j = 2 * i
                gather(sbuf.at[j + 1], rows1, sem1)
                gwait(rows0, sem0)
                scat(dbuf.at[j], rows0)
                gather(sbuf.at[j + 2], rows0, sem0)
                gwait(rows1, sem1)
                scat(dbuf.at[j + 1], rows1)
                return 0

            lax.fori_loop(0, IB // 2 - 1, body, 0)
            gather(sbuf.at[IB - 1], rows1, sem1)
            gwait(rows0, sem0)
            scat(dbuf.at[IB - 2], rows0)
            gwait(rows1, sem1)
            scat(dbuf.at[IB - 1], rows1)

            @pl.when(b + 2 < nb)
            def _():
                load_idx(b + 2, sbuf, dbuf)

    plsc.subcore_barrier()

    base = sid * ROWS_PER_TILE
    pltpu.sync_copy(agg_sh.at[pl.ds(base, K)], rows0)
    for r in range(ROWS_PER_TILE // K):
        if r + 1 < ROWS_PER_TILE // K:
            nxt = rows1 if r % 2 == 0 else rows0
            pltpu.async_copy(agg_sh.at[pl.ds(base + (r + 1) * K, K)], nxt, sem0)
        cur = rows0 if r % 2 == 0 else rows1
        pltpu.sync_copy(cur, agg_hbm.at[cid, pl.ds(base + r * K, K)])
        if r + 1 < ROWS_PER_TILE // K:
            nxt = rows1 if r % 2 == 0 else rows0
            pltpu.make_async_copy(sup_hbm.at[pl.ds(0, K)], nxt, sem0).wait()


def _final_body(agg_ref, degt_ref, b1_ref, wc_ref, bc_ref, z_ref, y_ref):
    deg = degt_ref[:, 1] + degt_ref[:, 3]
    inv = lax.rsqrt(jnp.clip(deg, 1.0, None))
    z = (agg_ref[0] + agg_ref[1]) * inv[:, None] + b1_ref[...][None, :]
    z_ref[...] = z
    y_ref[...] = jnp.dot(z, wc_ref[...],
                         preferred_element_type=jnp.float32) + bc_ref[0]


_FIN_BLK = 1000


def _final(agg, degt, b1, wc, bc):
    return pl.pallas_call(
        _final_body,
        grid=(N // _FIN_BLK,),
        in_specs=[
            pl.BlockSpec((NC, _FIN_BLK, F), lambda i: (0, i, 0)),
            pl.BlockSpec((_FIN_BLK, 2 * NC), lambda i: (i, 0)),
            pl.BlockSpec((F,), lambda i: (0,)),
            pl.BlockSpec((F, 1), lambda i: (0, 0)),
            pl.BlockSpec((1,), lambda i: (0,)),
        ],
        out_specs=[
            pl.BlockSpec((_FIN_BLK, F), lambda i: (i, 0)),
            pl.BlockSpec((_FIN_BLK, 1), lambda i: (i, 0)),
        ],
        out_shape=[
            jax.ShapeDtypeStruct((N, F), jnp.float32),
            jax.ShapeDtypeStruct((N, 1), jnp.float32),
        ],
    )(agg, degt, b1, wc, bc)


def kernel(g, x, W1, b1, Wc, bc):
    src = g[0]
    dst = g[1]
    pad = jnp.full((E_PAD - E,), NPAD - 1, jnp.int32)
    src_pad = jnp.concatenate([src, pad]).reshape(C_TOT, K)
    dst_pad = jnp.concatenate([dst, pad]).reshape(C_TOT, K)
    x_pad = jnp.concatenate([x, jnp.zeros((NPAD - N, F), jnp.float32)])

    deg = _sc_degrees(src_pad, dst_pad)
    deg = deg.reshape(2 * NC, NPAD)
    sup = _mm_scale(x_pad, W1, deg)
    agg = _sc_edge_pass(sup, src_pad, dst_pad)
    z, y = _final(agg, deg.T, b1, Wc, bc)
    return (z, y)


# speedup vs baseline: 1.0018x; 1.0018x over previous
import functools

import jax
import jax.numpy as jnp
from jax import lax
from jax.experimental import pallas as pl
from jax.experimental.pallas import tpu as pltpu
from jax.experimental.pallas import tpu_sc as plsc

N = 10000
E = 320000
F = 128

NC = 2
NS = 16
NW = NC * NS

K = 128
NPAD = 10240
ROWS_PER_TILE = NPAD // NS
CW = 80
EW = CW * K
E_PAD = EW * NW
C_TOT = E_PAD // K

_mesh = plsc.VectorSubcoreMesh(core_axis_name="c", subcore_axis_name="s")


def _zero_vmem_1d(ref, n):
    z = jnp.zeros((16,), jnp.float32)

    def body(i, _):
        ref[pl.ds(i * 16, 16)] = z
        return 0

    lax.fori_loop(0, n // 16, body, 0)


def _zero_vmem_2d(ref, rows, cols):
    z = jnp.zeros((16,), jnp.float32)

    def body(i, _):
        r = i // (cols // 16)
        c = lax.rem(i, cols // 16) * 16
        ref[r, pl.ds(c, 16)] = z
        return 0

    lax.fori_loop(0, rows * (cols // 16), body, 0)


@functools.partial(
    pl.kernel,
    out_type=jax.ShapeDtypeStruct((NC, 2, NPAD), jnp.float32),
    mesh=_mesh,
    scratch_types=[
        pltpu.VMEM((CW, K), jnp.int32),
        pltpu.VMEM((CW, K), jnp.int32),
        pltpu.VMEM((K,), jnp.float32),
        pltpu.VMEM((NPAD // NS,), jnp.float32),
        pltpu.VMEM_SHARED((NPAD,), jnp.float32),
        pltpu.VMEM_SHARED((NPAD,), jnp.float32),
        pltpu.SemaphoreType.DMA,
    ],
)
def _sc_degrees(src_hbm, dst_hbm, deg_out_hbm, sidx_v, didx_v, ones_v,
                stage_v, dout_sh, din_sh, sem):
    cid = lax.axis_index("c")
    sid = lax.axis_index("s")
    wid = sid * NC + cid
    seg = NPAD // NS

    pltpu.async_copy(src_hbm.at[pl.ds(wid * CW, CW)], sidx_v, sem)
    pltpu.async_copy(dst_hbm.at[pl.ds(wid * CW, CW)], didx_v, sem)

    _zero_vmem_1d(stage_v, seg)
    pltpu.sync_copy(stage_v, dout_sh.at[pl.ds(sid * seg, seg)])
    pltpu.sync_copy(stage_v, din_sh.at[pl.ds(sid * seg, seg)])
    one = jnp.ones((16,), jnp.float32)
    for i in range(K // 16):
        ones_v[pl.ds(i * 16, 16)] = one
    pltpu.make_async_copy(src_hbm.at[pl.ds(0, CW)], sidx_v, sem).wait()
    pltpu.make_async_copy(src_hbm.at[pl.ds(0, CW)], didx_v, sem).wait()
    plsc.subcore_barrier()

    BATCH = 8

    def batch(b, _):
        def fire(j, _):
            pltpu.async_copy(ones_v, dout_sh.at[sidx_v.at[b * BATCH + j]],
                             sem, add=True)
            pltpu.async_copy(ones_v, din_sh.at[didx_v.at[b * BATCH + j]],
                             sem, add=True)
            return 0

        lax.fori_loop(0, BATCH, fire, 0)

        def drain(j, _):
            pltpu.make_async_copy(ones_v, dout_sh.at[sidx_v.at[0]], sem).wait()
            pltpu.make_async_copy(ones_v, din_sh.at[didx_v.at[0]], sem).wait()
            return 0

        lax.fori_loop(0, BATCH, drain, 0)
        return 0

    lax.fori_loop(0, CW // BATCH, batch, 0)
    plsc.subcore_barrier()

    pltpu.sync_copy(dout_sh.at[pl.ds(sid * seg, seg)], stage_v)
    pltpu.sync_copy(stage_v, deg_out_hbm.at[cid, 0, pl.ds(sid * seg, seg)])
    pltpu.sync_copy(din_sh.at[pl.ds(sid * seg, seg)], stage_v)
    pltpu.sync_copy(stage_v, deg_out_hbm.at[cid, 1, pl.ds(sid * seg, seg)])


def _mm_scale_body(x_ref, w_ref, deg_ref, out_ref):
    deg = deg_ref[0, :] + deg_ref[2, :]
    inv = lax.rsqrt(jnp.clip(deg, 1.0, None))
    out_ref[...] = jnp.dot(x_ref[...], w_ref[...],
                           preferred_element_type=jnp.float32) * inv[:, None]


_MM_BLK = 1280


def _mm_scale(x_pad, w1, deg):
    return pl.pallas_call(
        _mm_scale_body,
        grid=(NPAD // _MM_BLK,),
        in_specs=[
            pl.BlockSpec((_MM_BLK, F), lambda i: (i, 0)),
            pl.BlockSpec((F, F), lambda i: (0, 0)),
            pl.BlockSpec((2 * NC, _MM_BLK), lambda i: (0, i)),
        ],
        out_specs=pl.BlockSpec((_MM_BLK, F), lambda i: (i, 0)),
        out_shape=jax.ShapeDtypeStruct((NPAD, F), jnp.float32),
    )(x_pad, w1, deg)


IB = 8
NB0 = 16
NB1 = 4
NBMAX = max(NB0, NB1)
assert NS * (NB0 + NB1) * IB == C_TOT


@functools.partial(
    pl.kernel,
    out_type=jax.ShapeDtypeStruct((NC, NPAD, F), jnp.float32),
    mesh=_mesh,
    scratch_types=[
        pltpu.VMEM((IB, K), jnp.int32),
        pltpu.VMEM((IB, K), jnp.int32),
        pltpu.VMEM((IB, K), jnp.int32),
        pltpu.VMEM((IB, K), jnp.int32),
        pltpu.VMEM((K, F), jnp.float32),
        pltpu.VMEM((K, F), jnp.float32),
        pltpu.VMEM_SHARED((NPAD, F), jnp.float32),
        pltpu.SemaphoreType.DMA,
        pltpu.SemaphoreType.DMA,
        pltpu.SemaphoreType.DMA,
    ],
)
def _sc_edge_pass(sup_hbm, src_hbm, dst_hbm, agg_hbm, sidxA, didxA, sidxB,
                  didxB, rows0, rows1, agg_sh, sem0, sem1, semi):
    cid = lax.axis_index("c")
    sid = lax.axis_index("s")
    nb = jnp.where(cid == 0, NB0, NB1)
    cbase = cid * (NS * NB0 * IB) + sid * nb * IB

    def load_idx(b, sbuf, dbuf):
        pltpu.async_copy(src_hbm.at[pl.ds(cbase + b * IB, IB)], sbuf, semi)
        pltpu.async_copy(dst_hbm.at[pl.ds(cbase + b * IB, IB)], dbuf, semi)

    def wait_idx(sbuf, dbuf):
        pltpu.make_async_copy(src_hbm.at[pl.ds(0, IB)], sbuf, semi).wait()
        pltpu.make_async_copy(src_hbm.at[pl.ds(0, IB)], dbuf, semi).wait()

    load_idx(0, sidxA, didxA)
    load_idx(1, sidxB, didxB)

    _zero_vmem_2d(rows0, K, F)
    for r in range(ROWS_PER_TILE // K):
        pltpu.sync_copy(rows0, agg_sh.at[pl.ds(sid * ROWS_PER_TILE + r * K, K)])
    plsc.subcore_barrier()

    def gather(idx_row, buf, sem):
        pltpu.async_copy(sup_hbm.at[idx_row], buf, sem)

    def gwait(buf, sem):
        pltpu.make_async_copy(sup_hbm.at[pl.ds(0, K)], buf, sem).wait()

    def scat(idx_row, buf):
        pltpu.sync_copy(buf, agg_sh.at[idx_row], add=True)

    for b in range(NBMAX):
        sbuf, dbuf = (sidxA, didxA) if b % 2 == 0 else (sidxB, didxB)

        @pl.when(b < nb)
        def _(b=b, sbuf=sbuf, dbuf=dbuf):
            wait_idx(sbuf, dbuf)
            gather(sbuf.at[0], rows0, sem0)

            def body(i, _):
                j = 2 * i
                gather(sbuf.at[j + 1], rows1, sem1)
                gwait(rows0, sem0)
                scat(dbuf.at[j], rows0)
                gather(sbuf.at[j + 2], rows0, sem0)
                gwait(rows1, sem1)
                scat(dbuf.at[j + 1], rows1)
                return 0

            lax.fori_loop(0, IB // 2 - 1, body, 0)
            gather(sbuf.at[IB - 1], rows1, sem1)
            gwait(rows0, sem0)
            scat(dbuf.at[IB - 2], rows0)
            gwait(rows1, sem1)
            scat(dbuf.at[IB - 1], rows1)

            @pl.when(b + 2 < nb)
            def _():
                load_idx(b + 2, sbuf, dbuf)

    plsc.subcore_barrier()

    base = sid * ROWS_PER_TILE
    pltpu.sync_copy(agg_sh.at[pl.ds(base, K)], rows0)
    for r in range(ROWS_PER_TILE // K):
        if r + 1 < ROWS_PER_TILE // K:
            nxt = rows1 if r % 2 == 0 else rows0
            pltpu.async_copy(agg_sh.at[pl.ds(base + (r + 1) * K, K)], nxt, sem0)
        cur = rows0 if r % 2 == 0 else rows1
        pltpu.sync_copy(cur, agg_hbm.at[cid, pl.ds(base + r * K, K)])
        if r + 1 < ROWS_PER_TILE // K:
            nxt = rows1 if r % 2 == 0 else rows0
            pltpu.make_async_copy(sup_hbm.at[pl.ds(0, K)], nxt, sem0).wait()


def _final_body(agg_ref, degt_ref, b1_ref, wc_ref, bc_ref, z_ref, y_ref):
    deg = degt_ref[:, 1] + degt_ref[:, 3]
    inv = lax.rsqrt(jnp.clip(deg, 1.0, None))
    z = (agg_ref[0] + agg_ref[1]) * inv[:, None] + b1_ref[...][None, :]
    z_ref[...] = z
    y_ref[...] = jnp.dot(z, wc_ref[...],
                         preferred_element_type=jnp.float32) + bc_ref[0]


_FIN_BLK = 1000


def _final(agg, degt, b1, wc, bc):
    return pl.pallas_call(
        _final_body,
        grid=(N // _FIN_BLK,),
        in_specs=[
            pl.BlockSpec((NC, _FIN_BLK, F), lambda i: (0, i, 0)),
            pl.BlockSpec((_FIN_BLK, 2 * NC), lambda i: (i, 0)),
            pl.BlockSpec((F,), lambda i: (0,)),
            pl.BlockSpec((F, 1), lambda i: (0, 0)),
            pl.BlockSpec((1,), lambda i: (0,)),
        ],
        out_specs=[
            pl.BlockSpec((_FIN_BLK, F), lambda i: (i, 0)),
            pl.BlockSpec((_FIN_BLK, 1), lambda i: (i, 0)),
        ],
        out_shape=[
            jax.ShapeDtypeStruct((N, F), jnp.float32),
            jax.ShapeDtypeStruct((N, 1), jnp.float32),
        ],
    )(agg, degt, b1, wc, bc)


def kernel(g, x, W1, b1, Wc, bc):
    src = g[0]
    dst = g[1]
    pad = jnp.full((E_PAD - E,), NPAD - 1, jnp.int32)
    src_pad = jnp.concatenate([src, pad]).reshape(C_TOT, K)
    dst_pad = jnp.concatenate([dst, pad]).reshape(C_TOT, K)
    x_pad = jnp.concatenate([x, jnp.zeros((NPAD - N, F), jnp.float32)])

    deg = _sc_degrees(src_pad, dst_pad)
    deg = deg.reshape(2 * NC, NPAD)
    sup = _mm_scale(x_pad, W1, deg)
    agg = _sc_edge_pass(sup, src_pad, dst_pad)
    z, y = _final(agg, deg.T, b1, Wc, bc)
    return (z, y)
